# baseline (speedup 1.0000x reference)
import jax
import jax.numpy as jnp
from jax import lax
from jax.experimental import pallas as pl
from jax.experimental.pallas import tpu as pltpu
from jax.experimental.pallas import tpu_sc as plsc

V = 1000
VP = 1024
N = 1024 * 50
NW = 32
NT = N // NW
C = 40
G = NT // C
NBUF = 3
LC = 80
LG = NT // LC


def _lse_body(tab_ref, lse_ref):
    x = tab_ref[...]
    m = jnp.max(x, axis=1, keepdims=True)
    s = jnp.sum(jnp.exp(x - m), axis=1, keepdims=True)
    lse_ref[...] = m + jnp.log(s)


def _reduce_body(part_ref, out_ref):
    out_ref[...] = (jnp.sum(part_ref[...]) * (1.0 / N)).reshape(1, 1)


def _sc_loss_body(idx_hbm, tgt_hbm, tabf_hbm, lse_hbm,
                  part_hbm,
                  idxt, tgtt, flatt, valt, lse_v, accv, psem):
    wid = lax.axis_index("s") * 2 + lax.axis_index("c")
    base = wid * NT

    pltpu.sync_copy(lse_hbm, lse_v)
    pltpu.sync_copy(idx_hbm.at[pl.ds(base, NT)], idxt)
    pltpu.sync_copy(tgt_hbm.at[pl.ds(base, NT)], tgtt)

    def flat_body(j, carry):
        sl = pl.ds(j * 16, 16)
        flatt[sl] = idxt[sl] * V + tgtt[sl]
        return carry

    lax.fori_loop(0, NT // 16, flat_body, 0)

    def fire_body(k, carry):
        sl = pl.ds(k * LC, LC)
        pltpu.async_copy(tabf_hbm.at[flatt.at[sl]], valt.at[sl], psem)
        return carry

    lax.fori_loop(0, LG, fire_body, 0)
    pltpu.make_async_copy(tabf_hbm.at[flatt], valt, psem).wait()

    zeros16 = jnp.zeros((16,), jnp.int32)

    def acc_body(j, acc):
        sl = pl.ds(j * 16, 16)
        lse_g = plsc.load_gather(lse_v, [idxt[sl], zeros16])
        return acc + (lse_g - valt[sl])

    acc = lax.fori_loop(0, NT // 16, acc_body, jnp.zeros((16,), jnp.float32))
    accv[...] = acc
    pltpu.sync_copy(accv, part_hbm.at[wid])


def _sc_gather_body(idx_hbm, table_hbm, out_hbm,
                    idxb0, idxb1, idxb2, rows0, rows1, rows2,
                    gsem0, gsem1, gsem2, ssem0, ssem1, ssem2):
    wid = lax.axis_index("s") * 2 + lax.axis_index("c")
    base = wid * NT
    bufs = ((idxb0, rows0, gsem0, ssem0),
            (idxb1, rows1, gsem1, ssem1),
            (idxb2, rows2, gsem2, ssem2))

    def start_gather(c, idxb, rows, gsem):
        pltpu.sync_copy(idx_hbm.at[pl.ds(base + c * C, C)], idxb)
        pltpu.async_copy(table_hbm.at[idxb], rows, gsem)

    for b in range(NBUF):
        start_gather(b, bufs[b][0], bufs[b][1], bufs[b][2])

    def main_body(i, carry):
        c_base = NBUF * i
        for b in range(NBUF):
            c = c_base + b
            idxb, rows, gsem, ssem = bufs[b]

            @pl.when(c < G)
            def _(c=c, idxb=idxb, rows=rows, gsem=gsem, ssem=ssem):
                pltpu.make_async_copy(table_hbm.at[idxb], rows, gsem).wait()
                pltpu.async_copy(rows, out_hbm.at[pl.ds(base + c * C, C)],
                                 ssem)

        for b in range(NBUF):
            c = c_base + b
            idxb, rows, gsem, ssem = bufs[b]

            @pl.when(c < G)
            def _(c=c, rows=rows, ssem=ssem):
                pltpu.make_async_copy(
                    rows, out_hbm.at[pl.ds(base + c * C, C)], ssem).wait()

            @pl.when(c + NBUF < G)
            def _(c=c, idxb=idxb, rows=rows, gsem=gsem):
                start_gather(c + NBUF, idxb, rows, gsem)

        return carry

    lax.fori_loop(0, (G + NBUF - 1) // NBUF, main_body, 0)


_MESH = dict(core_axis_name="c", subcore_axis_name="s")


def kernel(idx, target, table):
    idx_f = idx.reshape(-1).astype(jnp.int32)
    tgt_f = target.reshape(-1).astype(jnp.int32)
    table = table.astype(jnp.float32)

    lse = pl.pallas_call(
        _lse_body,
        out_shape=jax.ShapeDtypeStruct((V, 1), jnp.float32),
    )(table)

    loss_call = pl.kernel(
        _sc_loss_body,
        out_type=jax.ShapeDtypeStruct((NW, 16), jnp.float32),
        mesh=plsc.VectorSubcoreMesh(**_MESH),
        compiler_params=pltpu.CompilerParams(use_tc_tiling_on_sc=False,
                                             needs_layout_passes=False),
        scratch_types=[
            pltpu.VMEM((NT,), jnp.int32),
            pltpu.VMEM((NT,), jnp.int32),
            pltpu.VMEM((NT,), jnp.int32),
            pltpu.VMEM((NT,), jnp.float32),
            pltpu.VMEM((V, 1), jnp.float32),
            pltpu.VMEM((16,), jnp.float32),
            pltpu.SemaphoreType.DMA,
        ],
    )
    part = loss_call(idx_f, tgt_f, table.reshape(-1), lse)

    gather_call = pl.kernel(
        _sc_gather_body,
        out_type=jax.ShapeDtypeStruct((N, VP), jnp.float32),
        mesh=plsc.VectorSubcoreMesh(**_MESH),
        compiler_params=pltpu.CompilerParams(needs_layout_passes=False),
        scratch_types=[
            pltpu.VMEM((C,), jnp.int32),
            pltpu.VMEM((C,), jnp.int32),
            pltpu.VMEM((C,), jnp.int32),
            pltpu.VMEM((C, VP), jnp.float32),
            pltpu.VMEM((C, VP), jnp.float32),
            pltpu.VMEM((C, VP), jnp.float32),
            pltpu.SemaphoreType.DMA,
            pltpu.SemaphoreType.DMA,
            pltpu.SemaphoreType.DMA,
            pltpu.SemaphoreType.DMA,
            pltpu.SemaphoreType.DMA,
            pltpu.SemaphoreType.DMA,
        ],
    )
    table_pad = jnp.pad(table, ((0, 0), (0, VP - V)))
    out_pad = gather_call(idx_f, table_pad)
    logits2 = out_pad[:, :V]

    loss = pl.pallas_call(
        _reduce_body,
        out_shape=jax.ShapeDtypeStruct((1, 1), jnp.float32),
    )(part)

    return logits2, loss.reshape(())


# speedup vs baseline: 1.0150x; 1.0150x over previous
import jax
import jax.numpy as jnp
from jax import lax
from jax.experimental import pallas as pl
from jax.experimental.pallas import tpu as pltpu
from jax.experimental.pallas import tpu_sc as plsc

V = 1000
VP = 1024
N = 1024 * 50
NW = 32
NT = N // NW
C = 32
G = NT // C
LC = 80
LG = NT // LC


def _lse_body(tab_ref, lse_ref):
    x = tab_ref[...]
    m = jnp.max(x, axis=1, keepdims=True)
    s = jnp.sum(jnp.exp(x - m), axis=1, keepdims=True)
    lse_ref[...] = m + jnp.log(s)


def _reduce_body(part_ref, out_ref):
    out_ref[...] = (jnp.sum(part_ref[...]) * (1.0 / N)).reshape(1, 1)


def _sc_loss_body(idx_hbm, tgt_hbm, tabf_hbm, lse_hbm,
                  part_hbm,
                  idxt, tgtt, flatt, valt, lse_v, accv, psem):
    wid = lax.axis_index("s") * 2 + lax.axis_index("c")
    base = wid * NT

    pltpu.sync_copy(lse_hbm, lse_v)
    pltpu.sync_copy(idx_hbm.at[pl.ds(base, NT)], idxt)
    pltpu.sync_copy(tgt_hbm.at[pl.ds(base, NT)], tgtt)

    def flat_body(j, carry):
        sl = pl.ds(j * 16, 16)
        flatt[sl] = idxt[sl] * V + tgtt[sl]
        return carry

    lax.fori_loop(0, NT // 16, flat_body, 0)

    def fire_body(k, carry):
        sl = pl.ds(k * LC, LC)
        pltpu.async_copy(tabf_hbm.at[flatt.at[sl]], valt.at[sl], psem)
        return carry

    lax.fori_loop(0, LG, fire_body, 0)
    pltpu.make_async_copy(tabf_hbm.at[flatt], valt, psem).wait()

    zeros16 = jnp.zeros((16,), jnp.int32)

    def acc_body(j, acc):
        sl = pl.ds(j * 16, 16)
        lse_g = plsc.load_gather(lse_v, [idxt[sl], zeros16])
        return acc + (lse_g - valt[sl])

    acc = lax.fori_loop(0, NT // 16, acc_body, jnp.zeros((16,), jnp.float32))
    accv[...] = acc
    pltpu.sync_copy(accv, part_hbm.at[wid])


def _sc_gather_body(idx_hbm, table_hbm, out_hbm,
                    idxb0, idxb1, rows0, rows1,
                    gsem0, gsem1, ssem0, ssem1):
    wid = lax.axis_index("s") * 2 + lax.axis_index("c")
    base = wid * NT

    pltpu.sync_copy(idx_hbm.at[pl.ds(base, C)], idxb0)
    pltpu.async_copy(table_hbm.at[idxb0], rows0, gsem0)
    pltpu.sync_copy(idx_hbm.at[pl.ds(base + C, C)], idxb1)
    pltpu.async_copy(table_hbm.at[idxb1], rows1, gsem1)

    def main_body(i, carry):
        c0 = 2 * i
        c1 = 2 * i + 1
        pltpu.make_async_copy(table_hbm.at[idxb0], rows0, gsem0).wait()
        pltpu.async_copy(rows0, out_hbm.at[pl.ds(base + c0 * C, C)], ssem0)
        pltpu.make_async_copy(table_hbm.at[idxb1], rows1, gsem1).wait()
        pltpu.async_copy(rows1, out_hbm.at[pl.ds(base + c1 * C, C)], ssem1)
        pltpu.make_async_copy(rows0, out_hbm.at[pl.ds(base + c0 * C, C)],
                              ssem0).wait()

        @pl.when(c0 + 2 < G)
        def _():
            pltpu.sync_copy(idx_hbm.at[pl.ds(base + (c0 + 2) * C, C)], idxb0)
            pltpu.async_copy(table_hbm.at[idxb0], rows0, gsem0)

        pltpu.make_async_copy(rows1, out_hbm.at[pl.ds(base + c1 * C, C)],
                              ssem1).wait()

        @pl.when(c1 + 2 < G)
        def _():
            pltpu.sync_copy(idx_hbm.at[pl.ds(base + (c1 + 2) * C, C)], idxb1)
            pltpu.async_copy(table_hbm.at[idxb1], rows1, gsem1)

        return carry

    lax.fori_loop(0, G // 2, main_body, 0)


_MESH = dict(core_axis_name="c", subcore_axis_name="s")


def kernel(idx, target, table):
    idx_f = idx.reshape(-1).astype(jnp.int32)
    tgt_f = target.reshape(-1).astype(jnp.int32)
    table = table.astype(jnp.float32)

    lse = pl.pallas_call(
        _lse_body,
        out_shape=jax.ShapeDtypeStruct((V, 1), jnp.float32),
    )(table)

    loss_call = pl.kernel(
        _sc_loss_body,
        out_type=jax.ShapeDtypeStruct((NW, 16), jnp.float32),
        mesh=plsc.VectorSubcoreMesh(**_MESH),
        compiler_params=pltpu.CompilerParams(use_tc_tiling_on_sc=False,
                                             needs_layout_passes=False),
        scratch_types=[
            pltpu.VMEM((NT,), jnp.int32),
            pltpu.VMEM((NT,), jnp.int32),
            pltpu.VMEM((NT,), jnp.int32),
            pltpu.VMEM((NT,), jnp.float32),
            pltpu.VMEM((V, 1), jnp.float32),
            pltpu.VMEM((16,), jnp.float32),
            pltpu.SemaphoreType.DMA,
        ],
    )
    part = loss_call(idx_f, tgt_f, table.reshape(-1), lse)

    gather_call = pl.kernel(
        _sc_gather_body,
        out_type=jax.ShapeDtypeStruct((N, VP), jnp.float32),
        mesh=plsc.VectorSubcoreMesh(**_MESH),
        compiler_params=pltpu.CompilerParams(needs_layout_passes=False),
        scratch_types=[
            pltpu.VMEM((C,), jnp.int32),
            pltpu.VMEM((C,), jnp.int32),
            pltpu.VMEM((C, VP), jnp.float32),
            pltpu.VMEM((C, VP), jnp.float32),
            pltpu.SemaphoreType.DMA,
            pltpu.SemaphoreType.DMA,
            pltpu.SemaphoreType.DMA,
            pltpu.SemaphoreType.DMA,
        ],
    )
    table_pad = jnp.pad(table, ((0, 0), (0, VP - V)))
    out_pad = gather_call(idx_f, table_pad)
    logits2 = out_pad[:, :V]

    loss = pl.pallas_call(
        _reduce_body,
        out_shape=jax.ShapeDtypeStruct((1, 1), jnp.float32),
    )(part)

    return logits2, loss.reshape(())
